# baseline (speedup 1.0000x reference)
import functools

import jax
import jax.numpy as jnp
from jax import lax
from jax.experimental import pallas as pl
from jax.experimental.pallas import tpu as pltpu
from jax.experimental.pallas import tpu_sc as plsc

_THRESHOLDS = (0.5, 0.5, 0.5)

_N = 100000
_F = 128
_NREL = 3

_N_SC = 32000
_M_TC = _N - _N_SC
_TC_BLOCK = 4000

_NC = 2
_NS = 16
_NW = _NC * _NS
_ROWS_PER_WORKER = _N_SC // _NW
_CHUNK = 40
_NCHUNKS = _ROWS_PER_WORKER // _CHUNK


def _tc1_body(s_ref, n_ref, w_ref, o_ref):
    agg = s_ref[...]
    for r, t in enumerate(_THRESHOLDS):
        agg = agg + t * n_ref[r]
    o_ref[...] = jnp.maximum(
        jnp.dot(agg, w_ref[...], preferred_element_type=jnp.float32), 0.0
    )


def _tc2_body(prev_ref, a_ref, w_ref, o_ref):
    del prev_ref
    o_ref[...] = jnp.maximum(
        jnp.dot(a_ref[...], w_ref[...], preferred_element_type=jnp.float32), 0.0
    )


_DEPTH = 4


def _sc_agg_body(self_hbm, neigh_hbm, agg_hbm, bufs, in_sems, out_sems):
    wid = lax.axis_index("s") * _NC + lax.axis_index("c")

    def issue_in(k, slot):
        obase = wid * _ROWS_PER_WORKER + k * _CHUNK
        base = _M_TC + obase
        s_v, a_v, b_v, c_v = bufs[slot]
        sem = in_sems[slot]
        return [
            pltpu.async_copy(self_hbm.at[pl.ds(base, _CHUNK)], s_v, sem),
            pltpu.async_copy(neigh_hbm.at[pl.ds(0 * _N + base, _CHUNK)], a_v, sem),
            pltpu.async_copy(neigh_hbm.at[pl.ds(1 * _N + base, _CHUNK)], b_v, sem),
            pltpu.async_copy(neigh_hbm.at[pl.ds(2 * _N + base, _CHUNK)], c_v, sem),
        ]

    pending_in = {}
    pending_out = {}
    for k in range(min(_DEPTH - 1, _NCHUNKS)):
        pending_in[k % _DEPTH] = issue_in(k, k % _DEPTH)
    for k in range(_NCHUNKS):
        slot = k % _DEPTH
        kn = k + _DEPTH - 1
        if kn < _NCHUNKS:
            if kn % _DEPTH in pending_out:
                pending_out.pop(kn % _DEPTH).wait()
            pending_in[kn % _DEPTH] = issue_in(kn, kn % _DEPTH)
        for h in pending_in.pop(slot):
            h.wait()
        s_v, a_v, b_v, c_v = bufs[slot]

        def _row(i, carry):
            for j in range(_F // 16):
                sl = pl.ds(j * 16, 16)
                y = s_v[i, sl] + 0.5 * (a_v[i, sl] + b_v[i, sl] + c_v[i, sl])
                s_v[i, sl] = y
            return carry

        lax.fori_loop(0, _CHUNK, _row, 0)
        obase = wid * _ROWS_PER_WORKER + k * _CHUNK
        pending_out[slot] = pltpu.async_copy(
            s_v, agg_hbm.at[pl.ds(obase, _CHUNK)], out_sems[slot]
        )
    for h in pending_out.values():
        h.wait()


@functools.partial(
    pl.kernel,
    mesh=plsc.VectorSubcoreMesh(core_axis_name="c", subcore_axis_name="s"),
    out_type=jax.ShapeDtypeStruct((_N_SC, _F), jnp.float32),
    scratch_types=(
        [pltpu.VMEM((_CHUNK, _F), jnp.float32) for _ in range(4 * 4)]
        + [pltpu.SemaphoreType.DMA for _ in range(2 * 4)]
    ),
)
def _sc_aggregate(self_hbm, neigh_hbm, agg_hbm, *scr):
    bufs = tuple(tuple(scr[4 * d : 4 * d + 4]) for d in range(4))
    in_sems = tuple(scr[16 + d] for d in range(4))
    out_sems = tuple(scr[20 + d] for d in range(4))
    _sc_agg_body(self_hbm, neigh_hbm, agg_hbm, bufs, in_sems, out_sems)


def kernel(self_feats, neigh_feats, weight):
    n, f = self_feats.shape
    e = weight.shape[1]
    neigh3 = neigh_feats.reshape(_NREL, n, f)

    out1 = pl.pallas_call(
        _tc1_body,
        grid=(_M_TC // _TC_BLOCK,),
        in_specs=[
            pl.BlockSpec((_TC_BLOCK, f), lambda i: (i, 0)),
            pl.BlockSpec((_NREL, _TC_BLOCK, f), lambda i: (0, i, 0)),
            pl.BlockSpec((f, e), lambda i: (0, 0)),
        ],
        out_specs=pl.BlockSpec((_TC_BLOCK, e), lambda i: (i, 0)),
        out_shape=jax.ShapeDtypeStruct((n, e), jnp.float32),
    )(self_feats, neigh3, weight)

    agg = _sc_aggregate(self_feats, neigh_feats)

    off = _M_TC // _TC_BLOCK
    out = pl.pallas_call(
        _tc2_body,
        grid=(_N_SC // _TC_BLOCK,),
        in_specs=[
            pl.BlockSpec(memory_space=pl.ANY),
            pl.BlockSpec((_TC_BLOCK, f), lambda i: (i, 0)),
            pl.BlockSpec((f, e), lambda i: (0, 0)),
        ],
        out_specs=pl.BlockSpec((_TC_BLOCK, e), lambda i: (i + off, 0)),
        out_shape=jax.ShapeDtypeStruct((n, e), jnp.float32),
        input_output_aliases={0: 0},
    )(out1, agg, weight)
    return out


# speedup vs baseline: 1.4331x; 1.4331x over previous
import jax
import jax.numpy as jnp
from jax.experimental import pallas as pl
from jax.experimental.pallas import tpu as pltpu

_THRESHOLDS = (0.5, 0.5, 0.5)


def _body(s_ref, n_ref, w_ref, o_ref):
    agg = s_ref[...]
    for r, t in enumerate(_THRESHOLDS):
        agg = agg + t * n_ref[r]
    o_ref[...] = jnp.maximum(
        jnp.dot(agg, w_ref[...], preferred_element_type=jnp.float32), 0.0
    )


def kernel(self_feats, neigh_feats, weight):
    n, f = self_feats.shape
    e = weight.shape[1]
    nrel = neigh_feats.shape[0] // n
    block = 4000
    assert n % block == 0
    neigh3 = neigh_feats.reshape(nrel, n, f)
    return pl.pallas_call(
        _body,
        grid=(n // block,),
        in_specs=[
            pl.BlockSpec((block, f), lambda i: (i, 0)),
            pl.BlockSpec((nrel, block, f), lambda i: (0, i, 0)),
            pl.BlockSpec((f, e), lambda i: (0, 0)),
        ],
        out_specs=pl.BlockSpec((block, e), lambda i: (i, 0)),
        out_shape=jax.ShapeDtypeStruct((n, e), jnp.float32),
        compiler_params=pltpu.CompilerParams(
            dimension_semantics=("parallel",),
        ),
    )(self_feats, neigh3, weight)
